# baseline (speedup 1.0000x reference)
import jax
import jax.numpy as jnp
from jax.experimental import pallas as pl
from jax.experimental.pallas import tpu as pltpu

_BLOCK = 1024


def _bcast_body(emb_ref, out_ref, scratch_ref, sems):
    scratch_ref[...] = jnp.broadcast_to(emb_ref[...], scratch_ref.shape)
    n_copies = out_ref.shape[0] // _BLOCK
    copies = [
        pltpu.make_async_copy(
            scratch_ref,
            out_ref.at[pl.ds(i * _BLOCK, _BLOCK), :],
            sems.at[i],
        )
        for i in range(n_copies)
    ]
    for c in copies:
        c.start()
    for c in copies:
        c.wait()


def kernel(action_mask, embedding):
    batch = action_mask.shape[0]
    num_types, dim = embedding.shape
    flat = embedding.reshape(1, num_types * dim)
    n_copies = batch // _BLOCK
    out = pl.pallas_call(
        _bcast_body,
        in_specs=[pl.BlockSpec(memory_space=pltpu.MemorySpace.VMEM)],
        out_specs=pl.BlockSpec(memory_space=pltpu.MemorySpace.HBM),
        out_shape=jax.ShapeDtypeStruct((batch, num_types * dim), embedding.dtype),
        scratch_shapes=[
            pltpu.VMEM((_BLOCK, num_types * dim), embedding.dtype),
            pltpu.SemaphoreType.DMA((n_copies,)),
        ],
    )(flat)
    return out.reshape(batch, num_types, dim)


# speedup vs baseline: 1.9504x; 1.9504x over previous
import jax
import jax.numpy as jnp
from jax.experimental import pallas as pl
from jax.experimental.pallas import tpu as pltpu

_BLOCK = 1024


def _bcast_body(emb_ref, out_ref, scratch_ref, sems):
    scratch_ref[...] = jnp.broadcast_to(emb_ref[...][None], scratch_ref.shape)
    n_copies = out_ref.shape[0] // _BLOCK
    copies = [
        pltpu.make_async_copy(
            scratch_ref,
            out_ref.at[pl.ds(i * _BLOCK, _BLOCK), :, :],
            sems.at[i],
        )
        for i in range(n_copies)
    ]
    for c in copies:
        c.start()
    for c in copies:
        c.wait()


def kernel(action_mask, embedding):
    batch = action_mask.shape[0]
    num_types, dim = embedding.shape
    n_copies = batch // _BLOCK
    return pl.pallas_call(
        _bcast_body,
        in_specs=[pl.BlockSpec(memory_space=pltpu.MemorySpace.VMEM)],
        out_specs=pl.BlockSpec(memory_space=pltpu.MemorySpace.HBM),
        out_shape=jax.ShapeDtypeStruct((batch, num_types, dim), embedding.dtype),
        scratch_shapes=[
            pltpu.VMEM((_BLOCK, num_types, dim), embedding.dtype),
            pltpu.SemaphoreType.DMA((n_copies,)),
        ],
    )(embedding)
